# baseline (speedup 1.0000x reference)
import jax
import jax.numpy as jnp
from jax import lax
from jax.experimental import pallas as pl

N = 262144
NUM_CLASSES = 80

BLK = 2048
RB = N // BLK


def _body(cls_ref, lab_ref, obj_ref, tobj_ref, tbb_ref, gbb_ref, objc_ref,
          cls_out, ob_out):
    x = cls_ref[...]
    xT = jnp.transpose(x, (1, 0))
    lab = lab_ref[0]
    gobj = obj_ref[0]
    iota_c = lax.broadcasted_iota(jnp.int32, (NUM_CLASSES, BLK), 0)
    onehot = (iota_c == lab).astype(jnp.float32)
    e = jnp.exp(xT)
    s = jnp.sum(e, axis=0, keepdims=True)
    xt = jnp.sum(xT * onehot, axis=0, keepdims=True)
    logp = xt - jnp.log(s)
    p = jnp.exp(logp)
    f_cls = -(1.0 - p) * (1.0 - p) * logp
    m_cls = (gobj == 1).astype(jnp.float32)

    gobj_c = objc_ref[0]
    to = tobj_ref[...]
    eo = jnp.exp(to)
    so = eo[:, 0:1] + eo[:, 1:2]
    olab = jnp.clip(gobj_c, 0, 1)
    xo = jnp.where(olab == 0, to[:, 0:1], to[:, 1:2])
    logp_o = xo - jnp.log(so)
    po = jnp.exp(logp_o)
    f_obj = -(1.0 - po) * (1.0 - po) * logp_o
    m_obj = (gobj_c != -1).astype(jnp.float32)

    d = jnp.abs(tbb_ref[...] - gbb_ref[...])
    sl1 = jnp.where(d < 0.1, 0.5 * d * d / 0.1, d - 0.05)
    bb = jnp.sum(sl1, axis=1, keepdims=True)
    m_bb = (gobj_c == 1).astype(jnp.float32)

    @pl.when(pl.program_id(0) == 0)
    def _():
        cls_out[...] = jnp.zeros((1, BLK), jnp.float32)
        ob_out[...] = jnp.zeros((BLK, 2), jnp.float32)

    cls_out[...] += f_cls * m_cls
    ob_out[...] += jnp.concatenate([f_obj * m_obj, bb * m_bb], axis=1)


def _fused_loss(tcls, gcls3, gobj3, tobj, tbb, gbb, gobjc):
    return pl.pallas_call(
        _body,
        grid=(RB,),
        in_specs=[
            pl.BlockSpec((BLK, NUM_CLASSES), lambda i: (i, 0)),
            pl.BlockSpec((1, 1, BLK), lambda i: (i, 0, 0)),
            pl.BlockSpec((1, 1, BLK), lambda i: (i, 0, 0)),
            pl.BlockSpec((BLK, 2), lambda i: (i, 0)),
            pl.BlockSpec((BLK, 4), lambda i: (i, 0)),
            pl.BlockSpec((BLK, 4), lambda i: (i, 0)),
            pl.BlockSpec((1, BLK, 1), lambda i: (i, 0, 0)),
        ],
        out_specs=[
            pl.BlockSpec((1, BLK), lambda i: (0, 0)),
            pl.BlockSpec((BLK, 2), lambda i: (0, 0)),
        ],
        out_shape=[
            jax.ShapeDtypeStruct((1, BLK), jnp.float32),
            jax.ShapeDtypeStruct((BLK, 2), jnp.float32),
        ],
    )(tcls, gcls3, gobj3, tobj, tbb, gbb, gobjc)


def kernel(targets_bb, targets_cls, targets_obj, gt_targets_bb,
           gt_targets_cls, gt_targets_obj, w_objectness, w_class, w_bb, step):
    targets_cls = jnp.reshape(targets_cls, (-1, NUM_CLASSES))
    targets_bb = jnp.reshape(targets_bb, (-1, 4))
    targets_obj = jnp.reshape(targets_obj, (-1, 2))
    gbb = lax.stop_gradient(jnp.reshape(gt_targets_bb, (-1, 4)))
    gcls = jnp.reshape(gt_targets_cls, (-1,)).astype(jnp.int32)
    gobj = jnp.reshape(gt_targets_obj, (-1,)).astype(jnp.int32)

    gcls3 = jnp.reshape(gcls, (RB, 1, BLK))
    gobj3 = jnp.reshape(gobj, (RB, 1, BLK))
    gobjc = jnp.reshape(gobj, (RB, BLK, 1))

    cls_part, ob_part = _fused_loss(targets_cls, gcls3, gobj3,
                                    targets_obj, targets_bb, gbb, gobjc)

    num_anchors = jnp.float32(N)
    obj_loss = jnp.sum(ob_part[:, 0]) / num_anchors * 5000.0
    cls_loss = jnp.sum(cls_part) / num_anchors * 10000.0
    bb_loss = jnp.sum(ob_part[:, 1]) / num_anchors * 20000.0

    def _kendall(loss, w):
        return loss * jnp.exp(-w) + w

    return (_kendall(cls_loss, w_class),
            _kendall(obj_loss, w_objectness),
            _kendall(bb_loss, w_bb))
